# Initial kernel scaffold; baseline (speedup 1.0000x reference)
import jax
import jax.numpy as jnp
from jax.experimental import pallas as pl


def kernel(t, control):
    raise NotImplementedError("write your pallas kernel here")


# speedup vs baseline: 39.1039x; 39.1039x over previous
import jax
import jax.numpy as jnp
from jax import lax
from jax.experimental import pallas as pl
from jax.experimental.pallas import tpu as pltpu
from jax.experimental.pallas import tpu_sc as plsc

_STEPS = 1024
_CHANNELS = 256
_N = 65536
_NC = 2
_NS = 16
_NW = _NC * _NS
_PER_W = _N // _NW
_CHUNK = 128
_NCHUNK = _PER_W // _CHUNK
_L = 16


def _body(t_hbm, control_hbm, out_hbm,
          t_v, idx0_v, idx1_v, w1_v, row0_v, row1_v, out_v, sem0, sem1):
    wid = lax.axis_index("s") * _NC + lax.axis_index("c")
    base = wid * _PER_W

    pltpu.sync_copy(t_hbm.at[pl.ds(base, _PER_W)], t_v)

    def idx_body(g, carry):
        t16 = t_v[pl.ds(g * _L, _L)]
        pos = t16 * float(_STEPS - 1)
        i0 = lax.convert_element_type(pos, jnp.int32)
        i0 = jnp.maximum(jnp.minimum(i0, _STEPS - 2), 0)
        frac = pos - lax.convert_element_type(i0, jnp.float32)
        idx0_v[pl.ds(g * _L, _L)] = i0
        idx1_v[pl.ds(g * _L, _L)] = i0 + 1
        w1_v[pl.ds(g * _L, _L)] = frac
        return carry

    lax.fori_loop(0, _PER_W // _L, idx_body, 0)

    def chunk_body(k, carry):
        cp0 = pltpu.async_copy(
            control_hbm.at[idx0_v.at[pl.ds(k * _CHUNK, _CHUNK)]], row0_v, sem0)
        cp1 = pltpu.async_copy(
            control_hbm.at[idx1_v.at[pl.ds(k * _CHUNK, _CHUNK)]], row1_v, sem1)
        cp0.wait()
        cp1.wait()

        def t_body(j, inner):
            w16 = w1_v[pl.ds(k * _CHUNK + (j // _L) * _L, _L)]
            dn = lax.GatherDimensionNumbers(
                offset_dims=(), collapsed_slice_dims=(0,),
                start_index_map=(0,))
            w1 = lax.gather(
                w16, jnp.full((_L, 1), j % _L, jnp.int32), dn, (1,),
                mode=lax.GatherScatterMode.PROMISE_IN_BOUNDS)
            w0 = 1.0 - w1
            for c in range(_CHANNELS // _L):
                a = row0_v[j, pl.ds(c * _L, _L)]
                b = row1_v[j, pl.ds(c * _L, _L)]
                out_v[j, pl.ds(c * _L, _L)] = w0 * a + w1 * b
            return inner

        lax.fori_loop(0, _CHUNK, t_body, 0)
        pltpu.sync_copy(out_v, out_hbm.at[pl.ds(base + k * _CHUNK, _CHUNK)])
        return carry

    lax.fori_loop(0, _NCHUNK, chunk_body, 0)


def kernel(t, control):
    mesh = plsc.VectorSubcoreMesh(core_axis_name="c", subcore_axis_name="s")
    f = pl.kernel(
        _body,
        out_type=jax.ShapeDtypeStruct((_N, _CHANNELS), jnp.float32),
        mesh=mesh,
        scratch_types=[
            pltpu.VMEM((_PER_W,), jnp.float32),
            pltpu.VMEM((_PER_W,), jnp.int32),
            pltpu.VMEM((_PER_W,), jnp.int32),
            pltpu.VMEM((_PER_W,), jnp.float32),
            pltpu.VMEM((_CHUNK, _CHANNELS), jnp.float32),
            pltpu.VMEM((_CHUNK, _CHANNELS), jnp.float32),
            pltpu.VMEM((_CHUNK, _CHANNELS), jnp.float32),
            pltpu.SemaphoreType.DMA,
            pltpu.SemaphoreType.DMA,
        ],
    )
    return f(t, control)


# speedup vs baseline: 46.4085x; 1.1868x over previous
import jax
import jax.numpy as jnp
from jax import lax
from jax.experimental import pallas as pl
from jax.experimental.pallas import tpu as pltpu
from jax.experimental.pallas import tpu_sc as plsc

_STEPS = 1024
_CHANNELS = 256
_N = 65536
_NC = 2
_NS = 16
_NW = _NC * _NS
_PER_W = _N // _NW
_CHUNK = 64
_NCHUNK = _PER_W // _CHUNK
_NPAIR = _NCHUNK // 2
_L = 16


def _body(t_hbm, control_hbm, out_hbm,
          t_v, idx0_v, idx1_v, w1_v,
          r0a, r1a, r0b, r1b, oa, ob,
          sg0a, sg1a, sg0b, sg1b, soa, sob):
    wid = lax.axis_index("s") * _NC + lax.axis_index("c")
    base = wid * _PER_W

    pltpu.sync_copy(t_hbm.at[pl.ds(base, _PER_W)], t_v)

    def idx_body(g, carry):
        t16 = t_v[pl.ds(g * _L, _L)]
        pos = t16 * float(_STEPS - 1)
        i0 = lax.convert_element_type(pos, jnp.int32)
        i0 = jnp.maximum(jnp.minimum(i0, _STEPS - 2), 0)
        frac = pos - lax.convert_element_type(i0, jnp.float32)
        idx0_v[pl.ds(g * _L, _L)] = i0
        idx1_v[pl.ds(g * _L, _L)] = i0 + 1
        w1_v[pl.ds(g * _L, _L)] = frac
        return carry

    lax.fori_loop(0, _PER_W // _L, idx_body, 0)

    def g_descs(k, r0, r1, s0, s1):
        d0 = pltpu.make_async_copy(
            control_hbm.at[idx0_v.at[pl.ds(k * _CHUNK, _CHUNK)]], r0, s0)
        d1 = pltpu.make_async_copy(
            control_hbm.at[idx1_v.at[pl.ds(k * _CHUNK, _CHUNK)]], r1, s1)
        return d0, d1

    def g_issue(k, r0, r1, s0, s1):
        d0, d1 = g_descs(k, r0, r1, s0, s1)
        d0.start()
        d1.start()

    def g_wait(k, r0, r1, s0, s1):
        d0, d1 = g_descs(k, r0, r1, s0, s1)
        d0.wait()
        d1.wait()

    def o_desc(k, o, so):
        return pltpu.make_async_copy(
            o, out_hbm.at[pl.ds(base + k * _CHUNK, _CHUNK)], so)

    def combine(k, r0, r1, o):
        def t_body(j, inner):
            w16 = w1_v[pl.ds(k * _CHUNK + (j // _L) * _L, _L)]
            dn = lax.GatherDimensionNumbers(
                offset_dims=(), collapsed_slice_dims=(0,),
                start_index_map=(0,))
            w1 = lax.gather(
                w16, jnp.full((_L, 1), j % _L, jnp.int32), dn, (1,),
                mode=lax.GatherScatterMode.PROMISE_IN_BOUNDS)
            w0 = 1.0 - w1
            for c in range(_CHANNELS // _L):
                a = r0[j, pl.ds(c * _L, _L)]
                b = r1[j, pl.ds(c * _L, _L)]
                o[j, pl.ds(c * _L, _L)] = w0 * a + w1 * b
            return inner

        lax.fori_loop(0, _CHUNK, t_body, 0)

    g_issue(0, r0a, r1a, sg0a, sg1a)

    def pair_body(p, carry):
        k0 = 2 * p
        g_wait(k0, r0a, r1a, sg0a, sg1a)
        g_issue(k0 + 1, r0b, r1b, sg0b, sg1b)

        @pl.when(p > 0)
        def _():
            o_desc(k0 - 2, oa, soa).wait()

        combine(k0, r0a, r1a, oa)
        o_desc(k0, oa, soa).start()

        g_wait(k0 + 1, r0b, r1b, sg0b, sg1b)

        @pl.when(p + 1 < _NPAIR)
        def _():
            g_issue(k0 + 2, r0a, r1a, sg0a, sg1a)

        @pl.when(p > 0)
        def _():
            o_desc(k0 - 1, ob, sob).wait()

        combine(k0 + 1, r0b, r1b, ob)
        o_desc(k0 + 1, ob, sob).start()
        return carry

    lax.fori_loop(0, _NPAIR, pair_body, 0)

    o_desc(_NCHUNK - 2, oa, soa).wait()
    o_desc(_NCHUNK - 1, ob, sob).wait()


def kernel(t, control):
    mesh = plsc.VectorSubcoreMesh(core_axis_name="c", subcore_axis_name="s")
    f = pl.kernel(
        _body,
        out_type=jax.ShapeDtypeStruct((_N, _CHANNELS), jnp.float32),
        mesh=mesh,
        scratch_types=[
            pltpu.VMEM((_PER_W,), jnp.float32),
            pltpu.VMEM((_PER_W,), jnp.int32),
            pltpu.VMEM((_PER_W,), jnp.int32),
            pltpu.VMEM((_PER_W,), jnp.float32),
            pltpu.VMEM((_CHUNK, _CHANNELS), jnp.float32),
            pltpu.VMEM((_CHUNK, _CHANNELS), jnp.float32),
            pltpu.VMEM((_CHUNK, _CHANNELS), jnp.float32),
            pltpu.VMEM((_CHUNK, _CHANNELS), jnp.float32),
            pltpu.VMEM((_CHUNK, _CHANNELS), jnp.float32),
            pltpu.VMEM((_CHUNK, _CHANNELS), jnp.float32),
            pltpu.SemaphoreType.DMA,
            pltpu.SemaphoreType.DMA,
            pltpu.SemaphoreType.DMA,
            pltpu.SemaphoreType.DMA,
            pltpu.SemaphoreType.DMA,
            pltpu.SemaphoreType.DMA,
        ],
    )
    return f(t, control)
